# baseline (speedup 1.0000x reference)
import functools

import numpy as np
import jax
import jax.numpy as jnp
from jax import lax
from jax.experimental import pallas as pl
from jax.experimental.pallas import tpu as pltpu
import jax.experimental.pallas.tpu_sc as plsc

KS = 4
NNEG = 10
F = 512
B = 4
L = 2048

NCOL = KS * B * L
NWORK = 32
COLS_PER_W = NCOL // NWORK
CH = 4
NCH = COLS_PER_W // CH
NLANE = 16
NSEG = F // NLANE
NSEG2 = F // (2 * NLANE)


def _rotl(x, r):
    return ((x << np.uint32(r)) | (x >> np.uint32(32 - r))).astype(np.uint32)


def _threefry2x32(k0, k1, x0, x1):
    x0 = x0.astype(np.uint32)
    x1 = x1.astype(np.uint32)
    ks = [np.uint32(k0), np.uint32(k1), np.uint32(0)]
    ks[2] = np.uint32(ks[0] ^ ks[1] ^ np.uint32(0x1BD11BDA))
    rots = [(13, 15, 26, 6), (17, 29, 16, 24)]
    x0 = (x0 + ks[0]).astype(np.uint32)
    x1 = (x1 + ks[1]).astype(np.uint32)
    for blk in range(5):
        for r in rots[blk % 2]:
            x0 = (x0 + x1).astype(np.uint32)
            x1 = _rotl(x1, r)
            x1 = (x1 ^ x0).astype(np.uint32)
        x0 = (x0 + ks[(blk + 1) % 3]).astype(np.uint32)
        x1 = (x1 + ks[(blk + 2) % 3] + np.uint32(blk + 1)).astype(np.uint32)
    return x0, x1


def _np_uniform(k0, k1, size):
    i = np.arange(size, dtype=np.uint64)
    hi = (i >> np.uint64(32)).astype(np.uint32)
    lo = (i & np.uint64(0xFFFFFFFF)).astype(np.uint32)
    a, b = _threefry2x32(k0, k1, hi, lo)
    bits = a ^ b
    f = ((bits >> np.uint32(9)) | np.uint32(0x3F800000)).view(np.float32)
    return f - np.float32(1.0)


@functools.lru_cache(maxsize=1)
def _gather_indices():
    gidx = np.zeros((KS, B, L, NNEG), np.int32)
    k0, k1 = np.uint32(0), np.uint32(12345)
    for k in range(KS):
        time = L - k
        f0, f1 = _threefry2x32(k0, k1, np.array([0], np.uint32),
                               np.array([k], np.uint32))
        noise = _np_uniform(f0[0], f1[0],
                            time * B * time).reshape(time, B, time)
        idx = np.argsort(-noise, axis=-1, kind='stable')[:, :, :NNEG]
        for bb in range(B):
            gidx[k, bb, :time, :] = idx[:, bb, :] + bb * L
    return gidx.reshape(-1)


def _proj_body(c_ref, z_ref, w_ref, bias_ref, cs_ref, pos_ref):
    k = pl.program_id(0)
    cs = jnp.dot(c_ref[0], w_ref[0], preferred_element_type=jnp.float32)
    cs = cs + bias_ref[0]
    cs_ref[0, 0] = cs
    zsh = pltpu.roll(z_ref[0], L - k, 0)
    pos_ref[0, 0, 0] = jnp.sum(zsh * cs, axis=1)


def _proj_call(c_t, z_t, w_t, b3):
    return pl.pallas_call(
        _proj_body,
        grid=(KS, B),
        in_specs=[
            pl.BlockSpec((1, L, F), lambda k, b: (b, 0, 0)),
            pl.BlockSpec((1, L, F), lambda k, b: (b, 0, 0)),
            pl.BlockSpec((1, F, F), lambda k, b: (k, 0, 0)),
            pl.BlockSpec((1, 1, F), lambda k, b: (k, 0, 0)),
        ],
        out_specs=[
            pl.BlockSpec((1, 1, L, F), lambda k, b: (k, b, 0, 0)),
            pl.BlockSpec((1, 1, 1, L), lambda k, b: (k, b, 0, 0)),
        ],
        out_shape=[
            jax.ShapeDtypeStruct((KS, B, L, F), jnp.float32),
            jax.ShapeDtypeStruct((KS, B, 1, L), jnp.float32),
        ],
    )(c_t, z_t, w_t, b3)


NB = 2 * NLANE
DROWS_W = COLS_PER_W * NNEG // 4
FLUSH = 8


def _neg_body(z_hbm, cs_hbm, gidx_hbm, out_hbm,
              idx_all, zbuf0, zbuf1, cbuf0, cbuf1, obuf,
              semz0, semz1, semc0, semc1):
    wid = lax.axis_index("s") * 2 + lax.axis_index("c")
    base = wid * COLS_PER_W
    pltpu.sync_copy(gidx_hbm.at[wid], idx_all)

    def pair(ch, zbuf, cbuf, semz, semc):
        zsrc = z_hbm.at[idx_all.at[ch]]
        csrc = cs_hbm.at[pl.ds(base + ch * CH, CH)]
        return (zsrc, zbuf, semz), (csrc, cbuf, semc)

    def issue(ch, zbuf, cbuf, semz, semc):
        for args in pair(ch, zbuf, cbuf, semz, semc):
            pltpu.async_copy(*args)

    def wait(ch, zbuf, cbuf, semz, semc):
        for args in pair(ch, zbuf, cbuf, semz, semc):
            pltpu.make_async_copy(*args).wait()

    def compute(ch, zbuf, cbuf):
        slot = lax.rem(ch, FLUSH)
        for i in range(CH):
            cvec = [cbuf[i, pl.ds(NLANE * j, NLANE)] for j in range(NSEG)]
            for n in range(NNEG):
                r = i * NNEG + n
                acc = zbuf[r, pl.ds(0, NLANE)] * cvec[0]
                for j in range(1, NSEG):
                    acc = acc + zbuf[r, pl.ds(NLANE * j, NLANE)] * cvec[j]
                d = i * NNEG + n
                obuf[slot * (CH * NNEG // 8) + d // 8,
                     pl.ds((d % 8) * NLANE, NLANE)] = acc

    issue(0, zbuf0, cbuf0, semz0, semc0)

    def body2(g, carry):
        c0 = 2 * g
        c1 = 2 * g + 1
        issue(c1, zbuf1, cbuf1, semz1, semc1)
        wait(c0, zbuf0, cbuf0, semz0, semc0)
        compute(c0, zbuf0, cbuf0)
        c2 = lax.rem(c1 + 1, NCH)
        issue(c2, zbuf0, cbuf0, semz0, semc0)
        wait(c1, zbuf1, cbuf1, semz1, semc1)
        compute(c1, zbuf1, cbuf1)

        @pl.when(lax.rem(g, FLUSH // 2) == FLUSH // 2 - 1)
        def _():
            grp = g // (FLUSH // 2)
            nrows = FLUSH * CH * NNEG // 8
            off = pl.multiple_of(wid * (NCH * CH * NNEG // 8) + grp * nrows,
                                 8)
            pltpu.sync_copy(obuf, out_hbm.at[pl.ds(off, nrows)])

        return carry

    lax.fori_loop(0, NCH // 2, body2, 0)
    wait(0, zbuf0, cbuf0, semz0, semc0)


def _neg_call(z_flat, cs_flat, gidx):
    mesh = plsc.VectorSubcoreMesh(core_axis_name="c", subcore_axis_name="s")
    return pl.kernel(
        _neg_body,
        out_type=jax.ShapeDtypeStruct((NCOL * NNEG // 8, 128), jnp.float32),
        mesh=mesh,
        scratch_types=[
            pltpu.VMEM((NCH, CH * NNEG), jnp.int32),
            pltpu.VMEM((CH * NNEG, F), jnp.float32),
            pltpu.VMEM((CH * NNEG, F), jnp.float32),
            pltpu.VMEM((CH, F), jnp.float32),
            pltpu.VMEM((CH, F), jnp.float32),
            pltpu.VMEM((FLUSH * CH * NNEG // 8, 128), jnp.float32),
            pltpu.SemaphoreType.DMA,
            pltpu.SemaphoreType.DMA,
            pltpu.SemaphoreType.DMA,
            pltpu.SemaphoreType.DMA,
        ],
    )(z_flat, cs_flat, gidx)


def _logsig(x):
    return jnp.minimum(x, 0.0) - jnp.log(1.0 + jnp.exp(-jnp.abs(x)))


RPB = NCOL * NNEG // 8 // KS


def _loss_body(pos_ref, neg_ref, po_ref, no_ref):
    k = pl.program_id(0)
    timek = L - k
    p = pos_ref[0]
    tio = lax.broadcasted_iota(jnp.int32, (B, 1, L), 2)
    ps = jnp.sum(jnp.where(tio < timek, _logsig(p), 0.0))

    x = neg_ref[...]
    gi = lax.broadcasted_iota(jnp.int32, (128, 8), 0)
    gj = lax.broadcasted_iota(jnp.int32, (128, 8), 1)
    sel = (gi // NLANE == gj).astype(jnp.float32)
    dots = jnp.dot(x, sel, preferred_element_type=jnp.float32)
    ri = lax.broadcasted_iota(jnp.int32, (RPB, 8), 0)
    ci = lax.broadcasted_iota(jnp.int32, (RPB, 8), 1)
    dl = (ri * 8 + ci).astype(jnp.float32)
    bt = jnp.floor(dl * jnp.float32(1.0 / NNEG))
    t = bt - jnp.floor(bt * jnp.float32(1.0 / L)) * L
    ns = jnp.sum(jnp.where(t < timek, _logsig(-dots), 0.0))

    @pl.when(k == 0)
    def _():
        po_ref[...] = jnp.zeros_like(po_ref)
        no_ref[...] = jnp.zeros_like(no_ref)

    po_ref[...] += ps
    no_ref[...] += ns


def _loss_call(pos, negp):
    return pl.pallas_call(
        _loss_body,
        grid=(KS,),
        in_specs=[
            pl.BlockSpec((1, B, 1, L), lambda k: (k, 0, 0, 0)),
            pl.BlockSpec((RPB, 128), lambda k: (k, 0)),
        ],
        out_specs=[
            pl.BlockSpec((1, 128), lambda k: (0, 0)),
            pl.BlockSpec((1, 128), lambda k: (0, 0)),
        ],
        out_shape=[
            jax.ShapeDtypeStruct((1, 128), jnp.float32),
            jax.ShapeDtypeStruct((1, 128), jnp.float32),
        ],
    )(pos, negp)


def kernel(z, c, W, b):
    z_t = z.transpose(0, 2, 1)
    c_t = c.transpose(0, 2, 1)
    w_t = W.transpose(0, 2, 1)
    b3 = b[:, None, :]
    cs, pos = _proj_call(c_t, z_t, w_t, b3)
    gidx = jnp.asarray(_gather_indices())
    neg = _neg_call(z_t.reshape(B * L, F), cs.reshape(NCOL, F),
                    gidx.reshape(NWORK, NCH, CH * NNEG))
    po, no = _loss_call(pos, neg)
    total_pos = po[0, 0]
    total_neg = no[0, 0]
    total_loss = total_pos + NNEG * total_neg
    return (-total_pos, -total_neg, -total_loss)


# speedup vs baseline: 1.4254x; 1.4254x over previous
import functools

import numpy as np
import jax
import jax.numpy as jnp
from jax import lax
from jax.experimental import pallas as pl
from jax.experimental.pallas import tpu as pltpu
import jax.experimental.pallas.tpu_sc as plsc

KS = 4
NNEG = 10
F = 512
B = 4
L = 2048

NCOL = KS * B * L
NWORK = 32
COLS_PER_W = NCOL // NWORK
CH = 4
NCH = COLS_PER_W // CH
NLANE = 16
NSEG = F // NLANE
NSEG2 = F // (2 * NLANE)


def _rotl(x, r):
    return ((x << np.uint32(r)) | (x >> np.uint32(32 - r))).astype(np.uint32)


def _threefry2x32(k0, k1, x0, x1):
    x0 = x0.astype(np.uint32)
    x1 = x1.astype(np.uint32)
    ks = [np.uint32(k0), np.uint32(k1), np.uint32(0)]
    ks[2] = np.uint32(ks[0] ^ ks[1] ^ np.uint32(0x1BD11BDA))
    rots = [(13, 15, 26, 6), (17, 29, 16, 24)]
    x0 = (x0 + ks[0]).astype(np.uint32)
    x1 = (x1 + ks[1]).astype(np.uint32)
    for blk in range(5):
        for r in rots[blk % 2]:
            x0 = (x0 + x1).astype(np.uint32)
            x1 = _rotl(x1, r)
            x1 = (x1 ^ x0).astype(np.uint32)
        x0 = (x0 + ks[(blk + 1) % 3]).astype(np.uint32)
        x1 = (x1 + ks[(blk + 2) % 3] + np.uint32(blk + 1)).astype(np.uint32)
    return x0, x1


def _np_uniform(k0, k1, size):
    i = np.arange(size, dtype=np.uint64)
    hi = (i >> np.uint64(32)).astype(np.uint32)
    lo = (i & np.uint64(0xFFFFFFFF)).astype(np.uint32)
    a, b = _threefry2x32(k0, k1, hi, lo)
    bits = a ^ b
    f = ((bits >> np.uint32(9)) | np.uint32(0x3F800000)).view(np.float32)
    return f - np.float32(1.0)


@functools.lru_cache(maxsize=1)
def _gather_indices():
    gidx = np.zeros((KS, B, L, NNEG), np.int32)
    k0, k1 = np.uint32(0), np.uint32(12345)
    for k in range(KS):
        time = L - k
        f0, f1 = _threefry2x32(k0, k1, np.array([0], np.uint32),
                               np.array([k], np.uint32))
        noise = _np_uniform(f0[0], f1[0],
                            time * B * time).reshape(time, B, time)
        idx = np.argsort(-noise, axis=-1, kind='stable')[:, :, :NNEG]
        for bb in range(B):
            gidx[k, bb, :time, :] = idx[:, bb, :] + bb * L
    return gidx.reshape(-1)


def _proj_body(c_ref, z_ref, w_ref, bias_ref, cs_ref, pos_ref):
    k = pl.program_id(0)
    cs = jnp.dot(c_ref[0], w_ref[0], preferred_element_type=jnp.float32)
    cs = cs + bias_ref[0]
    cs_ref[0, 0] = cs
    zsh = pltpu.roll(z_ref[0], L - k, 0)
    pos_ref[0, 0, 0] = jnp.sum(zsh * cs, axis=1)


def _proj_call(c_t, z_t, w_t, b3):
    return pl.pallas_call(
        _proj_body,
        grid=(KS, B),
        in_specs=[
            pl.BlockSpec((1, L, F), lambda k, b: (b, 0, 0)),
            pl.BlockSpec((1, L, F), lambda k, b: (b, 0, 0)),
            pl.BlockSpec((1, F, F), lambda k, b: (k, 0, 0)),
            pl.BlockSpec((1, 1, F), lambda k, b: (k, 0, 0)),
        ],
        out_specs=[
            pl.BlockSpec((1, 1, L, F), lambda k, b: (k, b, 0, 0)),
            pl.BlockSpec((1, 1, 1, L), lambda k, b: (k, b, 0, 0)),
        ],
        out_shape=[
            jax.ShapeDtypeStruct((KS, B, L, F), jnp.float32),
            jax.ShapeDtypeStruct((KS, B, 1, L), jnp.float32),
        ],
    )(c_t, z_t, w_t, b3)


NB = 2 * NLANE
DROWS_W = COLS_PER_W * NNEG // 4
FLUSH = 8


def _neg_body(z_hbm, cs_hbm, gidx_hbm, out_hbm,
              idx_all, zbuf0, zbuf1, cbuf0, cbuf1, obuf,
              semz0, semz1, semc0, semc1):
    wid = lax.axis_index("s") * 2 + lax.axis_index("c")
    base = wid * COLS_PER_W
    pltpu.sync_copy(gidx_hbm.at[wid], idx_all)

    def pair(ch, zbuf, cbuf, semz, semc):
        zsrc = z_hbm.at[idx_all.at[ch]]
        csrc = cs_hbm.at[pl.ds(base + ch * CH, CH)]
        return (zsrc, zbuf, semz), (csrc, cbuf, semc)

    def issue(ch, zbuf, cbuf, semz, semc):
        for args in pair(ch, zbuf, cbuf, semz, semc):
            pltpu.async_copy(*args)

    def wait(ch, zbuf, cbuf, semz, semc):
        for args in pair(ch, zbuf, cbuf, semz, semc):
            pltpu.make_async_copy(*args).wait()

    def compute(ch, zbuf, cbuf):
        slot = lax.rem(ch, FLUSH)
        for i in range(CH):
            cvec = [cbuf[i, pl.ds(NLANE * j, NLANE)] for j in range(NSEG)]
            for n in range(NNEG):
                r = i * NNEG + n
                accs = [zbuf[r, pl.ds(NLANE * a, NLANE)] * cvec[a]
                        for a in range(4)]
                for j in range(4, NSEG):
                    a = j % 4
                    accs[a] = (accs[a]
                               + zbuf[r, pl.ds(NLANE * j, NLANE)] * cvec[j])
                acc = (accs[0] + accs[1]) + (accs[2] + accs[3])
                d = i * NNEG + n
                obuf[slot * (CH * NNEG // 8) + d // 8,
                     pl.ds((d % 8) * NLANE, NLANE)] = acc

    issue(0, zbuf0, cbuf0, semz0, semc0)

    def body2(g, carry):
        c0 = 2 * g
        c1 = 2 * g + 1
        issue(c1, zbuf1, cbuf1, semz1, semc1)
        wait(c0, zbuf0, cbuf0, semz0, semc0)
        compute(c0, zbuf0, cbuf0)
        c2 = lax.rem(c1 + 1, NCH)
        issue(c2, zbuf0, cbuf0, semz0, semc0)
        wait(c1, zbuf1, cbuf1, semz1, semc1)
        compute(c1, zbuf1, cbuf1)

        @pl.when(lax.rem(g, FLUSH // 2) == FLUSH // 2 - 1)
        def _():
            grp = g // (FLUSH // 2)
            nrows = FLUSH * CH * NNEG // 8
            off = pl.multiple_of(wid * (NCH * CH * NNEG // 8) + grp * nrows,
                                 8)
            pltpu.sync_copy(obuf, out_hbm.at[pl.ds(off, nrows)])

        return carry

    lax.fori_loop(0, NCH // 2, body2, 0)
    wait(0, zbuf0, cbuf0, semz0, semc0)


def _neg_call(z_flat, cs_flat, gidx):
    mesh = plsc.VectorSubcoreMesh(core_axis_name="c", subcore_axis_name="s")
    return pl.kernel(
        _neg_body,
        out_type=jax.ShapeDtypeStruct((NCOL * NNEG // 8, 128), jnp.float32),
        mesh=mesh,
        scratch_types=[
            pltpu.VMEM((NCH, CH * NNEG), jnp.int32),
            pltpu.VMEM((CH * NNEG, F), jnp.float32),
            pltpu.VMEM((CH * NNEG, F), jnp.float32),
            pltpu.VMEM((CH, F), jnp.float32),
            pltpu.VMEM((CH, F), jnp.float32),
            pltpu.VMEM((FLUSH * CH * NNEG // 8, 128), jnp.float32),
            pltpu.SemaphoreType.DMA,
            pltpu.SemaphoreType.DMA,
            pltpu.SemaphoreType.DMA,
            pltpu.SemaphoreType.DMA,
        ],
    )(z_flat, cs_flat, gidx)


def _logsig(x):
    return jnp.minimum(x, 0.0) - jnp.log(1.0 + jnp.exp(-jnp.abs(x)))


RPB = NCOL * NNEG // 8 // KS


def _loss_body(pos_ref, neg_ref, po_ref, no_ref):
    k = pl.program_id(0)
    timek = L - k
    p = pos_ref[0]
    tio = lax.broadcasted_iota(jnp.int32, (B, 1, L), 2)
    ps = jnp.sum(jnp.where(tio < timek, _logsig(p), 0.0))

    x = neg_ref[...]
    gi = lax.broadcasted_iota(jnp.int32, (128, 8), 0)
    gj = lax.broadcasted_iota(jnp.int32, (128, 8), 1)
    sel = (gi // NLANE == gj).astype(jnp.float32)
    dots = jnp.dot(x, sel, preferred_element_type=jnp.float32)
    ri = lax.broadcasted_iota(jnp.int32, (RPB, 8), 0)
    ci = lax.broadcasted_iota(jnp.int32, (RPB, 8), 1)
    dl = (ri * 8 + ci).astype(jnp.float32)
    bt = jnp.floor(dl * jnp.float32(1.0 / NNEG))
    t = bt - jnp.floor(bt * jnp.float32(1.0 / L)) * L
    ns = jnp.sum(jnp.where(t < timek, _logsig(-dots), 0.0))

    @pl.when(k == 0)
    def _():
        po_ref[...] = jnp.zeros_like(po_ref)
        no_ref[...] = jnp.zeros_like(no_ref)

    po_ref[...] += ps
    no_ref[...] += ns


def _loss_call(pos, negp):
    return pl.pallas_call(
        _loss_body,
        grid=(KS,),
        in_specs=[
            pl.BlockSpec((1, B, 1, L), lambda k: (k, 0, 0, 0)),
            pl.BlockSpec((RPB, 128), lambda k: (k, 0)),
        ],
        out_specs=[
            pl.BlockSpec((1, 128), lambda k: (0, 0)),
            pl.BlockSpec((1, 128), lambda k: (0, 0)),
        ],
        out_shape=[
            jax.ShapeDtypeStruct((1, 128), jnp.float32),
            jax.ShapeDtypeStruct((1, 128), jnp.float32),
        ],
    )(pos, negp)


def kernel(z, c, W, b):
    z_t = z.transpose(0, 2, 1)
    c_t = c.transpose(0, 2, 1)
    w_t = W.transpose(0, 2, 1)
    b3 = b[:, None, :]
    cs, pos = _proj_call(c_t, z_t, w_t, b3)
    gidx = jnp.asarray(_gather_indices())
    neg = _neg_call(z_t.reshape(B * L, F), cs.reshape(NCOL, F),
                    gidx.reshape(NWORK, NCH, CH * NNEG))
    po, no = _loss_call(pos, neg)
    total_pos = po[0, 0]
    total_neg = no[0, 0]
    total_loss = total_pos + NNEG * total_neg
    return (-total_pos, -total_neg, -total_loss)
